# baseline (speedup 1.0000x reference)
import functools

import jax
import jax.numpy as jnp
from jax import lax
from jax.experimental import pallas as pl
from jax.experimental.pallas import tpu as pltpu
from jax.experimental.pallas import tpu_sc as plsc

NEG = -1e30
CH = 128


def _sims_body(q_ref, k_ref, sims_ref, cm_ref, *, n_total, nt):
    i = pl.program_id(0)
    ncm = nt // CH
    qn_rows = q_ref.shape[0]
    last = (n_total - 1) // nt
    q = q_ref[...]
    qn = q / (jnp.sqrt(jnp.sum(q * q, axis=1, keepdims=True)) + 1e-8)
    kb = k_ref[...]
    kn = kb / (jnp.sqrt(jnp.sum(kb * kb, axis=1, keepdims=True)) + 1e-8)
    s = lax.dot_general(qn, kn, (((1,), (1,)), ((), ())),
                        preferred_element_type=jnp.float32)

    @pl.when(i == last)
    def _():
        col = i * nt + lax.broadcasted_iota(jnp.int32, (1, nt), 1)
        sm = jnp.where(col < n_total, s, NEG)
        for c in range(ncm):
            sims_ref[c * qn_rows:(c + 1) * qn_rows, :] = \
                sm[:, c * CH:(c + 1) * CH]
            cm_ref[0, :, c:c + 1] = jnp.max(sm[:, c * CH:(c + 1) * CH],
                                            axis=1, keepdims=True)

    @pl.when(i != last)
    def _():
        for c in range(ncm):
            sims_ref[c * qn_rows:(c + 1) * qn_rows, :] = \
                s[:, c * CH:(c + 1) * CH]
            cm_ref[0, :, c:c + 1] = jnp.max(s[:, c * CH:(c + 1) * CH],
                                            axis=1, keepdims=True)


def _chunk_select_body(cm_ref, sel_ref, *, m_real):
    qb, m = cm_ref.shape
    kk = sel_ref.shape[1]
    iota = lax.broadcasted_iota(jnp.int32, (qb, m), 1)
    vals = jnp.where(iota < m_real, cm_ref[...], -jnp.inf)
    big = jnp.int32(2 ** 30)
    selmask = jnp.zeros((qb, m), jnp.bool_)
    for _ in range(kk):
        mx = jnp.max(vals, axis=1, keepdims=True)
        c = jnp.min(jnp.where(vals == mx, iota, big), axis=1, keepdims=True)
        hit = iota == c
        selmask = selmask | hit
        vals = jnp.where(hit, -jnp.inf, vals)
    for j in range(kk):
        c = jnp.min(jnp.where(selmask, iota, big), axis=1, keepdims=True)
        sel_ref[:, j:j + 1] = c
        selmask = selmask & (iota != c)


def _topk_body(cand_ref, sel_ref, vals_ref, idx_ref):
    qb, kk = sel_ref.shape
    cand = cand_ref[...].reshape(qb, kk, CH)
    selb = sel_ref[...]
    p3 = (lax.broadcasted_iota(jnp.int32, (qb, kk, CH), 1) * CH
          + lax.broadcasted_iota(jnp.int32, (qb, kk, CH), 2))
    j_iota = lax.broadcasted_iota(jnp.int32, (qb, kk), 1)
    big = jnp.int32(2 ** 30)
    for it in range(kk):
        mx = jnp.max(jnp.max(cand, axis=2, keepdims=True), axis=1,
                     keepdims=True)
        pm = jnp.where(cand == mx, p3, big)
        p = jnp.min(jnp.min(pm, axis=2, keepdims=True), axis=1,
                    keepdims=True)
        p2 = p.reshape(qb, 1)
        jstar = lax.shift_right_logical(p2, 7)
        lane = p2 - (jstar * CH)
        selv = jnp.sum(jnp.where(j_iota == jstar, selb, 0), axis=1,
                       keepdims=True)
        vals_ref[:, it:it + 1] = mx.reshape(qb, 1)
        idx_ref[:, it:it + 1] = selv * CH + lane
        cand = jnp.where(p3 == p, -jnp.inf, cand)


def _precorrect_body(sh_ref, code_ref, rc_ref, out_ref):
    nb = sh_ref.shape[0]
    rrows = rc_ref.shape[0]
    c = code_ref[...]
    oh = (c == lax.broadcasted_iota(jnp.int32, (nb, rrows), 1)
          ).astype(jnp.float32)
    rcr = lax.dot_general(oh, rc_ref[...], (((1,), (0,)), ((), ())),
                          preferred_element_type=jnp.float32)
    out_ref[...] = sh_ref[...] - rcr


def _combine_body(g_ref, v_ref, qc_ref, rc_ref, t_ref, ss_ref, sb_ref,
                  out_ref):
    qb, dd = out_ref.shape
    kk = g_ref.shape[0] // qb
    g3 = g_ref[...].reshape(qb, kk, dd)
    v3 = v_ref[...].reshape(qb, kk, 1)
    t3 = t_ref[...].reshape(1, 1, dd)
    sc3 = v3 * t3
    smax = jnp.max(sc3, axis=1, keepdims=True)
    e = jnp.exp(sc3 - smax)
    den = jnp.sum(e, axis=1, keepdims=True)
    num = jnp.sum(e * g3, axis=1, keepdims=True)
    qc = qc_ref[...]
    oh = (qc == lax.broadcasted_iota(jnp.int32, (qb, rc_ref.shape[0]), 1)
          ).astype(jnp.float32)
    rcq = lax.dot_general(oh, rc_ref[...], (((1,), (0,)), ((), ())),
                          preferred_element_type=jnp.float32)
    out_ref[...] = (rcq + (num / den).reshape(qb, dd)) * ss_ref[...] \
        + sb_ref[...]


def _sc_gather_payload(table, ids):
    b_total = ids.shape[0]
    row_w = table.shape[1]
    info = plsc.get_sparse_core_info()
    nc = info.num_cores
    nw = nc * info.num_subcores
    bpw = b_total // nw
    half = bpw // 2
    mesh = plsc.VectorSubcoreMesh(core_axis_name="c", subcore_axis_name="s")

    @functools.partial(
        pl.kernel,
        out_type=jax.ShapeDtypeStruct((b_total, row_w), jnp.float32),
        mesh=mesh,
        scratch_types=[
            pltpu.VMEM((half,), jnp.int32),
            pltpu.VMEM((half,), jnp.int32),
            pltpu.VMEM((half, row_w), jnp.float32),
            pltpu.SemaphoreType.DMA,
        ],
    )
    def k(tab_hbm, ids_hbm, out_hbm, idx0, idx1, rows_v, sem):
        wid = lax.axis_index("s") * nc + lax.axis_index("c")
        base = wid * bpw
        for h in range(2):
            idxb = idx0 if h == 0 else idx1
            pltpu.sync_copy(ids_hbm.at[pl.ds(base + h * half, half)], idxb)
            pltpu.async_copy(tab_hbm.at[idxb], rows_v, sem).wait()
            pltpu.sync_copy(rows_v, out_hbm.at[pl.ds(base + h * half, half)])

    return k(table, ids)


def _sc_gather_chunks(table, sel_flat, q_n, kk):
    b_total = sel_flat.shape[0]
    row_w = table.shape[1]
    info = plsc.get_sparse_core_info()
    nc = info.num_cores
    nw = nc * info.num_subcores
    bpw = b_total // nw
    half = bpw // 2
    kshift = kk.bit_length() - 1
    assert (1 << kshift) == kk
    mesh = plsc.VectorSubcoreMesh(core_axis_name="c", subcore_axis_name="s")

    @functools.partial(
        pl.kernel,
        out_type=jax.ShapeDtypeStruct((b_total, row_w), jnp.float32),
        mesh=mesh,
        scratch_types=[
            pltpu.VMEM((bpw,), jnp.int32),
            pltpu.VMEM((half,), jnp.int32),
            pltpu.VMEM((half,), jnp.int32),
            pltpu.VMEM((half, row_w), jnp.float32),
            pltpu.SemaphoreType.DMA,
        ],
    )
    def k(tab_hbm, sel_hbm, out_hbm, sel_v, idx0, idx1, rows_v, sem):
        wid = lax.axis_index("s") * nc + lax.axis_index("c")
        base = wid * bpw
        pltpu.sync_copy(sel_hbm.at[pl.ds(base, bpw)], sel_v)
        lanes = lax.iota(jnp.int32, 16)
        for h in range(2):
            dst = idx0 if h == 0 else idx1
            for t in range(half // 16):
                off = h * half + t * 16
                pos = base + off + lanes
                qq = lax.shift_right_logical(pos, kshift)
                dst[pl.ds(t * 16, 16)] = sel_v[pl.ds(off, 16)] * q_n + qq
        for h in range(2):
            idxb = idx0 if h == 0 else idx1
            pltpu.async_copy(tab_hbm.at[idxb], rows_v, sem).wait()
            pltpu.sync_copy(rows_v, out_hbm.at[pl.ds(base + h * half, half)])

    return k(table, sel_flat)


def kernel(queries, keys, key_shifts, query_codes, key_codes, rc_table,
           temperature, shift_scale, shift_bias):
    q_n, d = queries.shape
    n = keys.shape[0]
    s_dim = key_shifts.shape[1]
    kk = 32
    nt = 512
    n_pad = ((n + nt - 1) // nt) * nt
    m_chunks = n_pad // CH

    query_codes = query_codes.astype(jnp.int32)
    key_codes = key_codes.astype(jnp.int32)

    grid_n = n_pad // nt
    ncm = nt // CH
    keys_p = jnp.pad(keys, ((0, n_pad - n), (0, 0)))
    sims, cmax3 = pl.pallas_call(
        functools.partial(_sims_body, n_total=n, nt=nt),
        grid=(grid_n,),
        in_specs=[
            pl.BlockSpec((q_n, d), lambda i: (0, 0)),
            pl.BlockSpec((nt, d), lambda i: (i, 0)),
        ],
        out_specs=[
            pl.BlockSpec((ncm * q_n, CH), lambda i: (i, 0)),
            pl.BlockSpec((1, q_n, ncm), lambda i: (i, 0, 0)),
        ],
        out_shape=[
            jax.ShapeDtypeStruct((m_chunks * q_n, CH), jnp.float32),
            jax.ShapeDtypeStruct((grid_n, q_n, ncm), jnp.float32),
        ],
    )(queries, keys_p)
    cmax = cmax3.transpose(1, 0, 2).reshape(q_n, m_chunks)

    qb2 = 256 if q_n % 256 == 0 else q_n
    sel = pl.pallas_call(
        functools.partial(_chunk_select_body, m_real=m_chunks),
        grid=(q_n // qb2,),
        in_specs=[pl.BlockSpec((qb2, m_chunks), lambda i: (i, 0))],
        out_specs=pl.BlockSpec((qb2, kk), lambda i: (i, 0)),
        out_shape=jax.ShapeDtypeStruct((q_n, kk), jnp.int32),
    )(cmax)

    cand = _sc_gather_chunks(sims, sel.reshape(-1), q_n, kk)

    qb4 = 128 if q_n % 128 == 0 else q_n
    vals, idx = pl.pallas_call(
        _topk_body,
        grid=(q_n // qb4,),
        in_specs=[
            pl.BlockSpec((qb4 * kk, CH), lambda i: (i, 0)),
            pl.BlockSpec((qb4, kk), lambda i: (i, 0)),
        ],
        out_specs=[
            pl.BlockSpec((qb4, kk), lambda i: (i, 0)),
            pl.BlockSpec((qb4, kk), lambda i: (i, 0)),
        ],
        out_shape=[
            jax.ShapeDtypeStruct((q_n, kk), jnp.float32),
            jax.ShapeDtypeStruct((q_n, kk), jnp.int32),
        ],
    )(cand, sel)

    dd = 128
    rrows = 32
    shifts16 = jnp.pad(key_shifts, ((0, 0), (0, dd - s_dim)))
    rc_p = jnp.pad(rc_table, ((0, rrows - rc_table.shape[0]),
                              (0, dd - s_dim)))
    nb5 = 2000 if n % 2000 == 0 else n
    table = pl.pallas_call(
        _precorrect_body,
        grid=(n // nb5,),
        in_specs=[
            pl.BlockSpec((nb5, dd), lambda i: (i, 0)),
            pl.BlockSpec((nb5, 1), lambda i: (i, 0)),
            pl.BlockSpec((rrows, dd), lambda i: (0, 0)),
        ],
        out_specs=pl.BlockSpec((nb5, dd), lambda i: (i, 0)),
        out_shape=jax.ShapeDtypeStruct((n, dd), jnp.float32),
    )(shifts16, key_codes.reshape(n, 1), rc_p)

    gath = _sc_gather_payload(table, idx.reshape(-1))

    t16 = jnp.pad(temperature, (0, dd - s_dim)).reshape(1, dd)
    ss16 = jnp.pad(shift_scale, (0, dd - s_dim)).reshape(1, dd)
    sb16 = jnp.pad(shift_bias, (0, dd - s_dim)).reshape(1, dd)
    qb7 = 128 if q_n % 128 == 0 else q_n
    out16 = pl.pallas_call(
        _combine_body,
        grid=(q_n // qb7,),
        in_specs=[
            pl.BlockSpec((qb7 * kk, dd), lambda i: (i, 0)),
            pl.BlockSpec((qb7 * kk, 1), lambda i: (i, 0)),
            pl.BlockSpec((qb7, 1), lambda i: (i, 0)),
            pl.BlockSpec((rrows, dd), lambda i: (0, 0)),
            pl.BlockSpec((1, dd), lambda i: (0, 0)),
            pl.BlockSpec((1, dd), lambda i: (0, 0)),
            pl.BlockSpec((1, dd), lambda i: (0, 0)),
        ],
        out_specs=pl.BlockSpec((qb7, dd), lambda i: (i, 0)),
        out_shape=jax.ShapeDtypeStruct((q_n, dd), jnp.float32),
    )(gath, vals.reshape(-1, 1), query_codes.reshape(q_n, 1), rc_p,
      t16, ss16, sb16)

    return out16[:, :s_dim]


# speedup vs baseline: 1.4140x; 1.4140x over previous
import functools

import jax
import jax.numpy as jnp
from jax import lax
from jax.experimental import pallas as pl
from jax.experimental.pallas import tpu as pltpu
from jax.experimental.pallas import tpu_sc as plsc

NEG = -1e30
CH = 128


def _sims_body(q_ref, k_ref, sims_ref, cm_ref, *, n_total, nt):
    i = pl.program_id(0)
    ncm = nt // CH
    qn_rows = q_ref.shape[0]
    last = (n_total - 1) // nt
    q = q_ref[...]
    qn = q / (jnp.sqrt(jnp.sum(q * q, axis=1, keepdims=True)) + 1e-8)
    kb = k_ref[...]
    kn = kb / (jnp.sqrt(jnp.sum(kb * kb, axis=1, keepdims=True)) + 1e-8)
    s = lax.dot_general(qn, kn, (((1,), (1,)), ((), ())),
                        preferred_element_type=jnp.float32)

    @pl.when(i == last)
    def _():
        col = i * nt + lax.broadcasted_iota(jnp.int32, (1, nt), 1)
        sm = jnp.where(col < n_total, s, NEG)
        for c in range(ncm):
            sims_ref[c * qn_rows:(c + 1) * qn_rows, :] = \
                sm[:, c * CH:(c + 1) * CH]
            cm_ref[0, :, c:c + 1] = jnp.max(sm[:, c * CH:(c + 1) * CH],
                                            axis=1, keepdims=True)

    @pl.when(i != last)
    def _():
        for c in range(ncm):
            sims_ref[c * qn_rows:(c + 1) * qn_rows, :] = \
                s[:, c * CH:(c + 1) * CH]
            cm_ref[0, :, c:c + 1] = jnp.max(s[:, c * CH:(c + 1) * CH],
                                            axis=1, keepdims=True)


def _chunk_select_body(cm_ref, sel_ref, *, m_real):
    qb, m = cm_ref.shape
    kk = sel_ref.shape[1]
    iota = lax.broadcasted_iota(jnp.int32, (qb, m), 1)
    vals = jnp.where(iota < m_real, cm_ref[...], -jnp.inf)
    big = jnp.int32(2 ** 30)
    selmask = jnp.zeros((qb, m), jnp.bool_)
    for _ in range(kk):
        mx = jnp.max(vals, axis=1, keepdims=True)
        c = jnp.min(jnp.where(vals == mx, iota, big), axis=1, keepdims=True)
        hit = iota == c
        selmask = selmask | hit
        vals = jnp.where(hit, -jnp.inf, vals)
    for j in range(kk):
        c = jnp.min(jnp.where(selmask, iota, big), axis=1, keepdims=True)
        sel_ref[:, j:j + 1] = c
        selmask = selmask & (iota != c)


def _topk_body(cand_ref, sel_ref, vals_ref, idx_ref):
    qb, w = cand_ref.shape
    kk = sel_ref.shape[1]
    cand = cand_ref[...]
    selb = sel_ref[...]
    p_iota = lax.broadcasted_iota(jnp.int32, (qb, w), 1)
    j_iota = lax.broadcasted_iota(jnp.int32, (qb, kk), 1)
    big = jnp.int32(2 ** 30)
    for it in range(kk):
        mx = jnp.max(cand, axis=1, keepdims=True)
        p = jnp.min(jnp.where(cand == mx, p_iota, big), axis=1, keepdims=True)
        jstar = lax.shift_right_logical(p, 7)
        lane = p - (jstar * CH)
        selv = jnp.sum(jnp.where(j_iota == jstar, selb, 0), axis=1,
                       keepdims=True)
        vals_ref[:, it:it + 1] = mx
        idx_ref[:, it:it + 1] = selv * CH + lane
        cand = jnp.where(p_iota == p, -jnp.inf, cand)


def _precorrect_body(sh_ref, code_ref, rc_ref, out_ref):
    nb = sh_ref.shape[0]
    rrows = rc_ref.shape[0]
    c = code_ref[...]
    oh = (c == lax.broadcasted_iota(jnp.int32, (nb, rrows), 1)
          ).astype(jnp.float32)
    rcr = lax.dot_general(oh, rc_ref[...], (((1,), (0,)), ((), ())),
                          preferred_element_type=jnp.float32)
    out_ref[...] = sh_ref[...] - rcr


def _combine_body(g_ref, v_ref, qc_ref, rc_ref, t_ref, ss_ref, sb_ref,
                  out_ref):
    qb, dd = out_ref.shape
    kk = g_ref.shape[0] // qb
    g3 = g_ref[...].reshape(qb, kk, dd)
    v3 = v_ref[...].reshape(qb, kk, 1)
    t3 = t_ref[...].reshape(1, 1, dd)
    sc3 = v3 * t3
    smax = jnp.max(sc3, axis=1, keepdims=True)
    e = jnp.exp(sc3 - smax)
    den = jnp.sum(e, axis=1, keepdims=True)
    num = jnp.sum(e * g3, axis=1, keepdims=True)
    qc = qc_ref[...]
    oh = (qc == lax.broadcasted_iota(jnp.int32, (qb, rc_ref.shape[0]), 1)
          ).astype(jnp.float32)
    rcq = lax.dot_general(oh, rc_ref[...], (((1,), (0,)), ((), ())),
                          preferred_element_type=jnp.float32)
    out_ref[...] = (rcq + (num / den).reshape(qb, dd)) * ss_ref[...] \
        + sb_ref[...]


def _sc_gather_payload(table, ids):
    b_total = ids.shape[0]
    row_w = table.shape[1]
    info = plsc.get_sparse_core_info()
    nc = info.num_cores
    nw = nc * info.num_subcores
    bpw = b_total // nw
    half = bpw // 2
    mesh = plsc.VectorSubcoreMesh(core_axis_name="c", subcore_axis_name="s")

    @functools.partial(
        pl.kernel,
        out_type=jax.ShapeDtypeStruct((b_total, row_w), jnp.float32),
        mesh=mesh,
        scratch_types=[
            pltpu.VMEM((half,), jnp.int32),
            pltpu.VMEM((half,), jnp.int32),
            pltpu.VMEM((half, row_w), jnp.float32),
            pltpu.SemaphoreType.DMA,
        ],
    )
    def k(tab_hbm, ids_hbm, out_hbm, idx0, idx1, rows_v, sem):
        wid = lax.axis_index("s") * nc + lax.axis_index("c")
        base = wid * bpw
        for h in range(2):
            idxb = idx0 if h == 0 else idx1
            pltpu.sync_copy(ids_hbm.at[pl.ds(base + h * half, half)], idxb)
            pltpu.async_copy(tab_hbm.at[idxb], rows_v, sem).wait()
            pltpu.sync_copy(rows_v, out_hbm.at[pl.ds(base + h * half, half)])

    return k(table, ids)


def _sc_gather_chunks(table, sel_flat, q_n, kk):
    b_total = sel_flat.shape[0]
    row_w = table.shape[1]
    info = plsc.get_sparse_core_info()
    nc = info.num_cores
    nw = nc * info.num_subcores
    bpw = b_total // nw
    half = bpw // 2
    kshift = kk.bit_length() - 1
    assert (1 << kshift) == kk
    mesh = plsc.VectorSubcoreMesh(core_axis_name="c", subcore_axis_name="s")

    @functools.partial(
        pl.kernel,
        out_type=jax.ShapeDtypeStruct((b_total, row_w), jnp.float32),
        mesh=mesh,
        scratch_types=[
            pltpu.VMEM((bpw,), jnp.int32),
            pltpu.VMEM((half,), jnp.int32),
            pltpu.VMEM((half,), jnp.int32),
            pltpu.VMEM((half, row_w), jnp.float32),
            pltpu.SemaphoreType.DMA,
        ],
    )
    def k(tab_hbm, sel_hbm, out_hbm, sel_v, idx0, idx1, rows_v, sem):
        wid = lax.axis_index("s") * nc + lax.axis_index("c")
        base = wid * bpw
        pltpu.sync_copy(sel_hbm.at[pl.ds(base, bpw)], sel_v)
        lanes = lax.iota(jnp.int32, 16)
        for h in range(2):
            dst = idx0 if h == 0 else idx1
            for t in range(half // 16):
                off = h * half + t * 16
                pos = base + off + lanes
                qq = lax.shift_right_logical(pos, kshift)
                dst[pl.ds(t * 16, 16)] = sel_v[pl.ds(off, 16)] * q_n + qq
        for h in range(2):
            idxb = idx0 if h == 0 else idx1
            pltpu.async_copy(tab_hbm.at[idxb], rows_v, sem).wait()
            pltpu.sync_copy(rows_v, out_hbm.at[pl.ds(base + h * half, half)])

    return k(table, sel_flat)


def kernel(queries, keys, key_shifts, query_codes, key_codes, rc_table,
           temperature, shift_scale, shift_bias):
    q_n, d = queries.shape
    n = keys.shape[0]
    s_dim = key_shifts.shape[1]
    kk = 32
    nt = 512
    n_pad = ((n + nt - 1) // nt) * nt
    m_chunks = n_pad // CH

    query_codes = query_codes.astype(jnp.int32)
    key_codes = key_codes.astype(jnp.int32)

    grid_n = n_pad // nt
    ncm = nt // CH
    keys_p = jnp.pad(keys, ((0, n_pad - n), (0, 0)))
    sims, cmax3 = pl.pallas_call(
        functools.partial(_sims_body, n_total=n, nt=nt),
        grid=(grid_n,),
        in_specs=[
            pl.BlockSpec((q_n, d), lambda i: (0, 0)),
            pl.BlockSpec((nt, d), lambda i: (i, 0)),
        ],
        out_specs=[
            pl.BlockSpec((ncm * q_n, CH), lambda i: (i, 0)),
            pl.BlockSpec((1, q_n, ncm), lambda i: (i, 0, 0)),
        ],
        out_shape=[
            jax.ShapeDtypeStruct((m_chunks * q_n, CH), jnp.float32),
            jax.ShapeDtypeStruct((grid_n, q_n, ncm), jnp.float32),
        ],
    )(queries, keys_p)
    cmax = cmax3.transpose(1, 0, 2).reshape(q_n, m_chunks)

    qb2 = 256 if q_n % 256 == 0 else q_n
    sel = pl.pallas_call(
        functools.partial(_chunk_select_body, m_real=m_chunks),
        grid=(q_n // qb2,),
        in_specs=[pl.BlockSpec((qb2, m_chunks), lambda i: (i, 0))],
        out_specs=pl.BlockSpec((qb2, kk), lambda i: (i, 0)),
        out_shape=jax.ShapeDtypeStruct((q_n, kk), jnp.int32),
    )(cmax)

    cand = _sc_gather_chunks(sims, sel.reshape(-1), q_n, kk)

    qb4 = 128 if q_n % 128 == 0 else q_n
    vals, idx = pl.pallas_call(
        _topk_body,
        grid=(q_n // qb4,),
        in_specs=[
            pl.BlockSpec((qb4, kk * CH), lambda i: (i, 0)),
            pl.BlockSpec((qb4, kk), lambda i: (i, 0)),
        ],
        out_specs=[
            pl.BlockSpec((qb4, kk), lambda i: (i, 0)),
            pl.BlockSpec((qb4, kk), lambda i: (i, 0)),
        ],
        out_shape=[
            jax.ShapeDtypeStruct((q_n, kk), jnp.float32),
            jax.ShapeDtypeStruct((q_n, kk), jnp.int32),
        ],
    )(cand.reshape(q_n, kk * CH), sel)

    dd = 128
    rrows = 32
    shifts16 = jnp.pad(key_shifts, ((0, 0), (0, dd - s_dim)))
    rc_p = jnp.pad(rc_table, ((0, rrows - rc_table.shape[0]),
                              (0, dd - s_dim)))
    nb5 = 2000 if n % 2000 == 0 else n
    table = pl.pallas_call(
        _precorrect_body,
        grid=(n // nb5,),
        in_specs=[
            pl.BlockSpec((nb5, dd), lambda i: (i, 0)),
            pl.BlockSpec((nb5, 1), lambda i: (i, 0)),
            pl.BlockSpec((rrows, dd), lambda i: (0, 0)),
        ],
        out_specs=pl.BlockSpec((nb5, dd), lambda i: (i, 0)),
        out_shape=jax.ShapeDtypeStruct((n, dd), jnp.float32),
    )(shifts16, key_codes.reshape(n, 1), rc_p)

    gath = _sc_gather_payload(table, idx.reshape(-1))

    t16 = jnp.pad(temperature, (0, dd - s_dim)).reshape(1, dd)
    ss16 = jnp.pad(shift_scale, (0, dd - s_dim)).reshape(1, dd)
    sb16 = jnp.pad(shift_bias, (0, dd - s_dim)).reshape(1, dd)
    qb7 = 128 if q_n % 128 == 0 else q_n
    out16 = pl.pallas_call(
        _combine_body,
        grid=(q_n // qb7,),
        in_specs=[
            pl.BlockSpec((qb7 * kk, dd), lambda i: (i, 0)),
            pl.BlockSpec((qb7 * kk, 1), lambda i: (i, 0)),
            pl.BlockSpec((qb7, 1), lambda i: (i, 0)),
            pl.BlockSpec((rrows, dd), lambda i: (0, 0)),
            pl.BlockSpec((1, dd), lambda i: (0, 0)),
            pl.BlockSpec((1, dd), lambda i: (0, 0)),
            pl.BlockSpec((1, dd), lambda i: (0, 0)),
        ],
        out_specs=pl.BlockSpec((qb7, dd), lambda i: (i, 0)),
        out_shape=jax.ShapeDtypeStruct((q_n, dd), jnp.float32),
    )(gath, vals.reshape(-1, 1), query_codes.reshape(q_n, 1), rc_p,
      t16, ss16, sb16)

    return out16[:, :s_dim]


# speedup vs baseline: 1.8559x; 1.3126x over previous
import functools

import jax
import jax.numpy as jnp
from jax import lax
from jax.experimental import pallas as pl
from jax.experimental.pallas import tpu as pltpu
from jax.experimental.pallas import tpu_sc as plsc

NEG = -1e30
CH = 128


def _sims_body(q_ref, k_ref, sh_ref, code_ref, rc_ref, sims_ref, cm_ref,
               tab_ref, *, n_total, nt):
    i = pl.program_id(0)
    ncm = nt // CH
    qn_rows = q_ref.shape[0]
    last = (n_total - 1) // nt
    q = q_ref[...]
    qn = q / (jnp.sqrt(jnp.sum(q * q, axis=1, keepdims=True)) + 1e-8)
    kb = k_ref[...]
    kn = kb / (jnp.sqrt(jnp.sum(kb * kb, axis=1, keepdims=True)) + 1e-8)
    s = lax.dot_general(qn, kn, (((1,), (1,)), ((), ())),
                        preferred_element_type=jnp.float32)

    c_codes = code_ref[...]
    oh = (c_codes == lax.broadcasted_iota(jnp.int32, (nt, rc_ref.shape[0]), 1)
          ).astype(jnp.float32)
    rcr = lax.dot_general(oh, rc_ref[...], (((1,), (0,)), ((), ())),
                          preferred_element_type=jnp.float32)
    tab_ref[...] = sh_ref[...] - rcr

    @pl.when(i == last)
    def _():
        col = i * nt + lax.broadcasted_iota(jnp.int32, (1, nt), 1)
        sm = jnp.where(col < n_total, s, NEG)
        for c in range(ncm):
            sims_ref[c * qn_rows:(c + 1) * qn_rows, :] = \
                sm[:, c * CH:(c + 1) * CH]
            cm_ref[0, :, c:c + 1] = jnp.max(sm[:, c * CH:(c + 1) * CH],
                                            axis=1, keepdims=True)

    @pl.when(i != last)
    def _():
        for c in range(ncm):
            sims_ref[c * qn_rows:(c + 1) * qn_rows, :] = \
                s[:, c * CH:(c + 1) * CH]
            cm_ref[0, :, c:c + 1] = jnp.max(s[:, c * CH:(c + 1) * CH],
                                            axis=1, keepdims=True)


def _chunk_select_body(cm_ref, sel_ref, *, m_real):
    qb, m = cm_ref.shape
    kk = sel_ref.shape[1]
    iota = lax.broadcasted_iota(jnp.int32, (qb, m), 1)
    vals = jnp.where(iota < m_real, cm_ref[...], -jnp.inf)
    big = jnp.int32(2 ** 30)
    selmask = jnp.zeros((qb, m), jnp.bool_)
    for _ in range(kk):
        mx = jnp.max(vals, axis=1, keepdims=True)
        c = jnp.min(jnp.where(vals == mx, iota, big), axis=1, keepdims=True)
        hit = iota == c
        selmask = selmask | hit
        vals = jnp.where(hit, -jnp.inf, vals)
    for j in range(kk):
        c = jnp.min(jnp.where(selmask, iota, big), axis=1, keepdims=True)
        sel_ref[:, j:j + 1] = c
        selmask = selmask & (iota != c)


def _topk_body(cand_ref, sel_ref, vals_ref, idx_ref):
    qb, w = cand_ref.shape
    kk = sel_ref.shape[1]
    cand = cand_ref[...]
    selb = sel_ref[...]
    p_iota = lax.broadcasted_iota(jnp.int32, (qb, w), 1)
    j_iota = lax.broadcasted_iota(jnp.int32, (qb, kk), 1)
    big = jnp.int32(2 ** 30)
    for it in range(kk):
        mx = jnp.max(cand, axis=1, keepdims=True)
        p = jnp.min(jnp.where(cand == mx, p_iota, big), axis=1, keepdims=True)
        jstar = lax.shift_right_logical(p, 7)
        lane = p - (jstar * CH)
        selv = jnp.sum(jnp.where(j_iota == jstar, selb, 0), axis=1,
                       keepdims=True)
        vals_ref[:, it:it + 1] = mx
        idx_ref[:, it:it + 1] = selv * CH + lane
        cand = jnp.where(p_iota == p, -jnp.inf, cand)


def _combine_body(g_ref, v_ref, qc_ref, rc_ref, t_ref, ss_ref, sb_ref,
                  out_ref):
    qb, dd = out_ref.shape
    kk = g_ref.shape[0] // qb
    g3 = g_ref[...].reshape(qb, kk, dd)
    v3 = v_ref[...].reshape(qb, kk, 1)
    t3 = t_ref[...].reshape(1, 1, dd)
    sc3 = v3 * t3
    smax = jnp.max(sc3, axis=1, keepdims=True)
    e = jnp.exp(sc3 - smax)
    den = jnp.sum(e, axis=1, keepdims=True)
    num = jnp.sum(e * g3, axis=1, keepdims=True)
    qc = qc_ref[...]
    oh = (qc == lax.broadcasted_iota(jnp.int32, (qb, rc_ref.shape[0]), 1)
          ).astype(jnp.float32)
    rcq = lax.dot_general(oh, rc_ref[...], (((1,), (0,)), ((), ())),
                          preferred_element_type=jnp.float32)
    out_ref[...] = (rcq + (num / den).reshape(qb, dd)) * ss_ref[...] \
        + sb_ref[...]


def _sc_gather_payload(table, ids):
    b_total = ids.shape[0]
    row_w = table.shape[1]
    info = plsc.get_sparse_core_info()
    nc = info.num_cores
    nw = nc * info.num_subcores
    bpw = b_total // nw
    half = bpw // 2
    mesh = plsc.VectorSubcoreMesh(core_axis_name="c", subcore_axis_name="s")

    @functools.partial(
        pl.kernel,
        out_type=jax.ShapeDtypeStruct((b_total, row_w), jnp.float32),
        mesh=mesh,
        scratch_types=[
            pltpu.VMEM((half,), jnp.int32),
            pltpu.VMEM((half,), jnp.int32),
            pltpu.VMEM((half, row_w), jnp.float32),
            pltpu.SemaphoreType.DMA,
        ],
    )
    def k(tab_hbm, ids_hbm, out_hbm, idx0, idx1, rows_v, sem):
        wid = lax.axis_index("s") * nc + lax.axis_index("c")
        base = wid * bpw
        for h in range(2):
            idxb = idx0 if h == 0 else idx1
            pltpu.sync_copy(ids_hbm.at[pl.ds(base + h * half, half)], idxb)
            pltpu.async_copy(tab_hbm.at[idxb], rows_v, sem).wait()
            pltpu.sync_copy(rows_v, out_hbm.at[pl.ds(base + h * half, half)])

    return k(table, ids)


def _sc_gather_chunks(table, sel_flat, q_n, kk):
    b_total = sel_flat.shape[0]
    row_w = table.shape[1]
    info = plsc.get_sparse_core_info()
    nc = info.num_cores
    nw = nc * info.num_subcores
    bpw = b_total // nw
    half = bpw // 2
    kshift = kk.bit_length() - 1
    assert (1 << kshift) == kk
    mesh = plsc.VectorSubcoreMesh(core_axis_name="c", subcore_axis_name="s")

    @functools.partial(
        pl.kernel,
        out_type=jax.ShapeDtypeStruct((b_total, row_w), jnp.float32),
        mesh=mesh,
        scratch_types=[
            pltpu.VMEM((bpw,), jnp.int32),
            pltpu.VMEM((half,), jnp.int32),
            pltpu.VMEM((half,), jnp.int32),
            pltpu.VMEM((half, row_w), jnp.float32),
            pltpu.SemaphoreType.DMA,
        ],
    )
    def k(tab_hbm, sel_hbm, out_hbm, sel_v, idx0, idx1, rows_v, sem):
        wid = lax.axis_index("s") * nc + lax.axis_index("c")
        base = wid * bpw
        pltpu.sync_copy(sel_hbm.at[pl.ds(base, bpw)], sel_v)
        lanes = lax.iota(jnp.int32, 16)
        for h in range(2):
            dst = idx0 if h == 0 else idx1
            for t in range(half // 16):
                off = h * half + t * 16
                pos = base + off + lanes
                qq = lax.shift_right_logical(pos, kshift)
                dst[pl.ds(t * 16, 16)] = sel_v[pl.ds(off, 16)] * q_n + qq
        for h in range(2):
            idxb = idx0 if h == 0 else idx1
            pltpu.async_copy(tab_hbm.at[idxb], rows_v, sem).wait()
            pltpu.sync_copy(rows_v, out_hbm.at[pl.ds(base + h * half, half)])

    return k(table, sel_flat)


def kernel(queries, keys, key_shifts, query_codes, key_codes, rc_table,
           temperature, shift_scale, shift_bias):
    q_n, d = queries.shape
    n = keys.shape[0]
    s_dim = key_shifts.shape[1]
    kk = 32
    nt = 3584
    n_pad = ((n + nt - 1) // nt) * nt
    m_chunks = n_pad // CH
    dd = 128
    rrows = 32

    query_codes = query_codes.astype(jnp.int32)
    key_codes = key_codes.astype(jnp.int32)

    grid_n = n_pad // nt
    ncm = nt // CH
    keys_p = jnp.pad(keys, ((0, n_pad - n), (0, 0)))
    shifts_p = jnp.pad(key_shifts, ((0, n_pad - n), (0, dd - s_dim)))
    codes_p = jnp.pad(key_codes, (0, n_pad - n)).reshape(n_pad, 1)
    rc_p = jnp.pad(rc_table, ((0, rrows - rc_table.shape[0]),
                              (0, dd - s_dim)))
    sims, cmax3, table = pl.pallas_call(
        functools.partial(_sims_body, n_total=n, nt=nt),
        grid=(grid_n,),
        in_specs=[
            pl.BlockSpec((q_n, d), lambda i: (0, 0)),
            pl.BlockSpec((nt, d), lambda i: (i, 0)),
            pl.BlockSpec((nt, dd), lambda i: (i, 0)),
            pl.BlockSpec((nt, 1), lambda i: (i, 0)),
            pl.BlockSpec((rrows, dd), lambda i: (0, 0)),
        ],
        out_specs=[
            pl.BlockSpec((ncm * q_n, CH), lambda i: (i, 0)),
            pl.BlockSpec((1, q_n, ncm), lambda i: (i, 0, 0)),
            pl.BlockSpec((nt, dd), lambda i: (i, 0)),
        ],
        out_shape=[
            jax.ShapeDtypeStruct((m_chunks * q_n, CH), jnp.float32),
            jax.ShapeDtypeStruct((grid_n, q_n, ncm), jnp.float32),
            jax.ShapeDtypeStruct((n_pad, dd), jnp.float32),
        ],
    )(queries, keys_p, shifts_p, codes_p, rc_p)
    cmax = cmax3.transpose(1, 0, 2).reshape(q_n, m_chunks)

    qb2 = q_n
    sel = pl.pallas_call(
        functools.partial(_chunk_select_body, m_real=m_chunks),
        grid=(q_n // qb2,),
        in_specs=[pl.BlockSpec((qb2, m_chunks), lambda i: (i, 0))],
        out_specs=pl.BlockSpec((qb2, kk), lambda i: (i, 0)),
        out_shape=jax.ShapeDtypeStruct((q_n, kk), jnp.int32),
    )(cmax)

    cand = _sc_gather_chunks(sims, sel.reshape(-1), q_n, kk)

    qb4 = 512 if q_n % 512 == 0 else q_n
    vals, idx = pl.pallas_call(
        _topk_body,
        grid=(q_n // qb4,),
        in_specs=[
            pl.BlockSpec((qb4, kk * CH), lambda i: (i, 0)),
            pl.BlockSpec((qb4, kk), lambda i: (i, 0)),
        ],
        out_specs=[
            pl.BlockSpec((qb4, kk), lambda i: (i, 0)),
            pl.BlockSpec((qb4, kk), lambda i: (i, 0)),
        ],
        out_shape=[
            jax.ShapeDtypeStruct((q_n, kk), jnp.float32),
            jax.ShapeDtypeStruct((q_n, kk), jnp.int32),
        ],
    )(cand.reshape(q_n, kk * CH), sel)

    gath = _sc_gather_payload(table, idx.reshape(-1))

    t16 = jnp.pad(temperature, (0, dd - s_dim)).reshape(1, dd)
    ss16 = jnp.pad(shift_scale, (0, dd - s_dim)).reshape(1, dd)
    sb16 = jnp.pad(shift_bias, (0, dd - s_dim)).reshape(1, dd)
    qb7 = 128 if q_n % 128 == 0 else q_n
    out16 = pl.pallas_call(
        _combine_body,
        grid=(q_n // qb7,),
        in_specs=[
            pl.BlockSpec((qb7 * kk, dd), lambda i: (i, 0)),
            pl.BlockSpec((qb7 * kk, 1), lambda i: (i, 0)),
            pl.BlockSpec((qb7, 1), lambda i: (i, 0)),
            pl.BlockSpec((rrows, dd), lambda i: (0, 0)),
            pl.BlockSpec((1, dd), lambda i: (0, 0)),
            pl.BlockSpec((1, dd), lambda i: (0, 0)),
            pl.BlockSpec((1, dd), lambda i: (0, 0)),
        ],
        out_specs=pl.BlockSpec((qb7, dd), lambda i: (i, 0)),
        out_shape=jax.ShapeDtypeStruct((q_n, dd), jnp.float32),
    )(gath, vals.reshape(-1, 1), query_codes.reshape(q_n, 1), rc_p,
      t16, ss16, sb16)

    return out16[:, :s_dim]
